# baseline (speedup 1.0000x reference)
import functools

import jax
import jax.numpy as jnp
from jax import lax
from jax.experimental import pallas as pl
from jax.experimental.pallas import tpu as pltpu
from jax.experimental.pallas import tpu_sc as plsc


@functools.lru_cache(maxsize=None)
def _build_bcast_kernel(batch, seq_len, d_model, dtype_name):
    dtype = jnp.dtype(dtype_name)
    info = plsc.get_sparse_core_info()
    num_cores, num_subcores = info.num_cores, info.num_subcores
    num_workers = num_cores * num_subcores
    assert seq_len % num_workers == 0, seq_len
    rows_per_w = seq_len // num_workers

    mesh = plsc.VectorSubcoreMesh(core_axis_name="c", subcore_axis_name="s")

    @functools.partial(
        pl.kernel,
        mesh=mesh,
        out_type=jax.ShapeDtypeStruct((batch, seq_len, d_model), dtype),
        scratch_types=[pltpu.SemaphoreType.DMA],
    )
    def bcast(table_hbm, out_hbm, sem):
        wid = lax.axis_index("s") * num_cores + lax.axis_index("c")
        base = wid * rows_per_w
        copies = [
            pltpu.async_copy(
                table_hbm.at[pl.ds(base, rows_per_w)],
                out_hbm.at[b, pl.ds(base, rows_per_w)],
                sem,
            )
            for b in range(batch)
        ]
        for c in copies:
            c.wait()

    return bcast


def kernel(x, pos_table):
    batch, seq_len, d_model = x.shape
    fn = _build_bcast_kernel(batch, seq_len, d_model, str(pos_table.dtype))
    return fn(pos_table[:seq_len])


# speedup vs baseline: 32.0911x; 32.0911x over previous
import functools

import jax
import jax.numpy as jnp
from jax import lax
from jax.experimental import pallas as pl
from jax.experimental.pallas import tpu as pltpu
from jax.experimental.pallas import tpu_sc as plsc


@functools.lru_cache(maxsize=None)
def _build_bcast_kernel(batch, seq_len, d_model, dtype_name):
    dtype = jnp.dtype(dtype_name)
    info = plsc.get_sparse_core_info()
    num_cores, num_subcores = info.num_cores, info.num_subcores
    num_workers = num_cores * num_subcores
    assert seq_len % num_workers == 0, seq_len
    rows_per_w = seq_len // num_workers

    mesh = plsc.VectorSubcoreMesh(core_axis_name="c", subcore_axis_name="s")

    @functools.partial(
        pl.kernel,
        mesh=mesh,
        out_type=jax.ShapeDtypeStruct((batch, seq_len, d_model), dtype),
        scratch_types=[
            pltpu.VMEM((seq_len // num_workers, d_model), dtype),
            pltpu.SemaphoreType.DMA,
        ],
    )
    def bcast(table_hbm, out_hbm, rows_v, sem):
        wid = lax.axis_index("s") * num_cores + lax.axis_index("c")
        base = wid * rows_per_w
        pltpu.sync_copy(table_hbm.at[pl.ds(base, rows_per_w)], rows_v)
        copies = [
            pltpu.async_copy(
                rows_v,
                out_hbm.at[b, pl.ds(base, rows_per_w)],
                sem,
            )
            for b in range(batch)
        ]
        for c in copies:
            c.wait()

    return bcast


def kernel(x, pos_table):
    batch, seq_len, d_model = x.shape
    fn = _build_bcast_kernel(batch, seq_len, d_model, str(pos_table.dtype))
    return fn(pos_table[:seq_len])
